# baseline (speedup 1.0000x reference)
import functools

import jax
import jax.numpy as jnp
from jax import lax
from jax.experimental import pallas as pl
from jax.experimental.pallas import tpu as pltpu
from jax.experimental.pallas import tpu_sc as plsc

_HP = 304
_NTILE = 320
_EC = 32
_NW = 32


def _node_kernel(h_ref, xyz_ref, wo1_ref, bo1_ref, wo2_ref, bo2_ref,
                 we1c_ref, we1h_ref, be1_ref, g_ref, b_ref):
    h = h_ref[...]
    xyz = xyz_ref[...]
    t = jnp.maximum(
        jnp.dot(h, wo1_ref[...], preferred_element_type=jnp.float32)
        + bo1_ref[...], 0.0)
    off = jnp.dot(t, wo2_ref[...], preferred_element_type=jnp.float32) + bo2_ref[...]
    c = xyz + off
    we1c = we1c_ref[...]
    g_ref[...] = (jnp.dot(h, we1h_ref[...], preferred_element_type=jnp.float32)
                  + jnp.dot(xyz, we1c, preferred_element_type=jnp.float32)
                  + be1_ref[...])
    b_ref[...] = jnp.dot(c, we1c, preferred_element_type=jnp.float32)


def _edge_kernel(e1_ref, we2_ref, be2_ref, e2_ref):
    e2_ref[...] = jnp.maximum(
        jnp.dot(e1_ref[...], we2_ref[...], preferred_element_type=jnp.float32)
        + be2_ref[...], 0.0)


def _update_kernel(h_ref, agg_ref, wu1_ref, bu1_ref, wu2_ref, bu2_ref, out_ref):
    t = jnp.maximum(
        jnp.dot(agg_ref[...], wu1_ref[...], preferred_element_type=jnp.float32)
        + bu1_ref[...], 0.0)
    out_ref[...] = h_ref[...] + (
        jnp.dot(t, wu2_ref[...], preferred_element_type=jnp.float32)
        + bu2_ref[...])


def _make_scatter_max(e_num, n_pad):
    mesh = plsc.VectorSubcoreMesh(core_axis_name="c", subcore_axis_name="s")

    nstage = 64

    @functools.partial(
        pl.kernel, mesh=mesh,
        out_type=jax.ShapeDtypeStruct((n_pad, _HP), jnp.float32),
        scratch_types=[
            pltpu.VMEM((nstage, _HP), jnp.float32),
            pltpu.VMEM((_EC, _HP), jnp.float32),
            pltpu.VMEM((_EC + 16,), jnp.int32),
            pltpu.VMEM((_NW + 16,), jnp.int32),
            pltpu.VMEM((_NW + 16,), jnp.int32),
        ],
    )
    def scatter_max(e2_hbm, dst_hbm, elo_hbm, ehi_hbm, out_hbm,
                    stage, ebuf, dbuf, elo_v, ehi_v):
        nc = 2
        wid = lax.axis_index("s") * nc + lax.axis_index("c")
        nlo = pl.multiple_of(wid * _NTILE, _NTILE)
        pltpu.sync_copy(elo_hbm, elo_v.at[pl.ds(0, _NW)])
        pltpu.sync_copy(ehi_hbm, ehi_v.at[pl.ds(0, _NW)])
        elo = elo_v[pl.ds(wid, 16)][0]
        ehi = ehi_v[pl.ds(wid, 16)][0]

        def zero_stage():
            def zero_row(r, _):
                for k in range(_HP // 16):
                    stage[r, pl.ds(k * 16, 16)] = jnp.zeros((16,), jnp.float32)
                return 0
            lax.fori_loop(0, nstage, zero_row, 0)

        def flush(cb):
            pltpu.sync_copy(
                stage, out_hbm.at[pl.ds(pl.multiple_of(nlo + cb, 8), nstage)])
            zero_stage()

        zero_stage()
        nchunks = (ehi - elo + _EC - 1) // _EC

        def chunk_body(j, cur_base):
            start = pl.multiple_of(jnp.minimum(elo + j * _EC, e_num - _EC), 8)
            pltpu.sync_copy(e2_hbm.at[pl.ds(start, _EC)], ebuf)
            pltpu.sync_copy(dst_hbm.at[pl.ds(start, _EC)],
                            dbuf.at[pl.ds(0, _EC)])

            def edge_body(i, cb):
                row = dbuf[pl.ds(i, 16)][0] - nlo
                tgt = jnp.clip((row // nstage) * nstage, 0, _NTILE - nstage)
                nflush = jnp.maximum((tgt - cb) // nstage, 0)

                def adv_body(m, c):
                    flush(c)
                    return c + nstage
                cb = lax.fori_loop(0, nflush, adv_body, cb)

                @pl.when((row >= cb) & (row < cb + nstage))
                def _():
                    r = row - cb
                    for k in range(_HP // 16):
                        sl = pl.ds(k * 16, 16)
                        stage[r, sl] = jnp.maximum(stage[r, sl],
                                                   ebuf[i, sl])
                return cb
            return lax.fori_loop(0, _EC, edge_body, cur_base)
        cur_base = lax.fori_loop(0, nchunks, chunk_body, 0)

        def tail_body(m, c):
            flush(c)
            return c + nstage
        lax.fori_loop(0, (_NTILE - cur_base) // nstage, tail_body, cur_base)

    return scatter_max


def kernel(x, xyz, edge_index, Wo1, bo1, Wo2, bo2, We1, be1, We2, be2,
           Wu1, bu1, Wu2, bu2):
    src = edge_index[0]
    dst = edge_index[1]
    n, d = x.shape
    e_num = src.shape[0]
    num_layers = Wo1.shape[0]
    n_pad = _NW * _NTILE

    order = jnp.argsort(dst)
    dst_s = dst[order]
    src_s = src[order]
    bounds = jnp.searchsorted(dst_s, jnp.arange(_NW + 1, dtype=jnp.int32) * _NTILE)
    elo8 = ((bounds[:_NW] // 8) * 8).astype(jnp.int32)
    ehi = bounds[1:].astype(jnp.int32)

    pad_c = _HP - We2.shape[-1]
    We1p = jnp.pad(We1, ((0, 0), (0, 0), (0, pad_c)))
    be1p = jnp.pad(be1, ((0, 0), (0, pad_c)))
    We2p = jnp.pad(We2, ((0, 0), (0, pad_c), (0, pad_c)))
    be2p = jnp.pad(be2, ((0, 0), (0, pad_c)))
    Wu1p = jnp.pad(Wu1, ((0, 0), (0, pad_c), (0, pad_c)))
    bu1p = jnp.pad(bu1, ((0, 0), (0, pad_c)))
    Wu2p = jnp.pad(Wu2, ((0, 0), (0, pad_c), (0, 0)))

    BN = 400
    BE = 2000

    h = x
    full = lambda shape: pl.BlockSpec(shape, lambda i: (0,) * len(shape))

    node_call = pl.pallas_call(
        _node_kernel,
        grid=(n // BN,),
        in_specs=[
            pl.BlockSpec((BN, d), lambda i: (i, 0)),
            pl.BlockSpec((BN, 3), lambda i: (i, 0)),
            full(Wo1.shape[1:]), full((1, bo1.shape[-1])),
            full(Wo2.shape[1:]), full((1, 3)),
            full((3, _HP)), full((d, _HP)), full((1, _HP)),
        ],
        out_specs=[
            pl.BlockSpec((BN, _HP), lambda i: (i, 0)),
            pl.BlockSpec((BN, _HP), lambda i: (i, 0)),
        ],
        out_shape=[
            jax.ShapeDtypeStruct((n, _HP), jnp.float32),
            jax.ShapeDtypeStruct((n, _HP), jnp.float32),
        ],
    )

    edge_call = pl.pallas_call(
        _edge_kernel,
        grid=(e_num // BE,),
        in_specs=[
            pl.BlockSpec((BE, _HP), lambda i: (i, 0)),
            full((_HP, _HP)), full((1, _HP)),
        ],
        out_specs=pl.BlockSpec((BE, _HP), lambda i: (i, 0)),
        out_shape=jax.ShapeDtypeStruct((e_num, _HP), jnp.float32),
    )

    update_call = pl.pallas_call(
        _update_kernel,
        grid=(n // BN,),
        in_specs=[
            pl.BlockSpec((BN, d), lambda i: (i, 0)),
            pl.BlockSpec((BN, _HP), lambda i: (i, 0)),
            full((_HP, _HP)), full((1, _HP)),
            full((_HP, d)), full((1, d)),
        ],
        out_specs=pl.BlockSpec((BN, d), lambda i: (i, 0)),
        out_shape=jax.ShapeDtypeStruct((n, d), jnp.float32),
    )

    scatter_call = _make_scatter_max(e_num, n_pad)

    for l in range(num_layers):
        g, b = node_call(
            h, xyz, Wo1[l], bo1[l][None], Wo2[l], bo2[l][None],
            We1p[l, :3], We1p[l, 3:], be1p[l][None])
        e1 = jnp.maximum(g[src_s] - b[dst_s], 0.0)
        e2 = edge_call(e1, We2p[l], be2p[l][None])
        agg = scatter_call(e2, dst_s, elo8, ehi)[:n]
        h = update_call(h, agg, Wu1p[l], bu1p[l][None], Wu2p[l], bu2[l][None])
    return h


# speedup vs baseline: 1.2882x; 1.2882x over previous
import functools

import jax
import jax.numpy as jnp
from jax import lax
from jax.experimental import pallas as pl
from jax.experimental.pallas import tpu as pltpu
from jax.experimental.pallas import tpu_sc as plsc

_HP = 304
_HPG = 384
_NTILE = 320
_EC = 32
_NW = 32


def _node_kernel(h_ref, xyz_ref, wo1_ref, bo1_ref, wo2_ref, bo2_ref,
                 we1c_ref, we1h_ref, be1_ref, g_ref, b_ref):
    h = h_ref[...]
    xyz = xyz_ref[...]
    t = jnp.maximum(
        jnp.dot(h, wo1_ref[...], preferred_element_type=jnp.float32)
        + bo1_ref[...], 0.0)
    off = jnp.dot(t, wo2_ref[...], preferred_element_type=jnp.float32) + bo2_ref[...]
    c = xyz + off
    we1c = we1c_ref[...]
    g_ref[...] = (jnp.dot(h, we1h_ref[...], preferred_element_type=jnp.float32)
                  + jnp.dot(xyz, we1c, preferred_element_type=jnp.float32)
                  + be1_ref[...])
    b_ref[...] = jnp.dot(c, we1c, preferred_element_type=jnp.float32)


def _edge_kernel(e1_ref, we2_ref, be2_ref, e2_ref):
    e2_ref[...] = jnp.maximum(
        jnp.dot(e1_ref[...], we2_ref[...], preferred_element_type=jnp.float32)
        + be2_ref[...], 0.0)


def _update_kernel(h_ref, agg_ref, wu1_ref, bu1_ref, wu2_ref, bu2_ref, out_ref):
    t = jnp.maximum(
        jnp.dot(agg_ref[...], wu1_ref[...], preferred_element_type=jnp.float32)
        + bu1_ref[...], 0.0)
    out_ref[...] = h_ref[...] + (
        jnp.dot(t, wu2_ref[...], preferred_element_type=jnp.float32)
        + bu2_ref[...])


def _make_gather_sub(e_num):
    span = e_num // _NW
    nch = (span + _EC - 1) // _EC
    nch2 = nch + (nch % 2)
    mesh = plsc.VectorSubcoreMesh(core_axis_name="c", subcore_axis_name="s")

    @functools.partial(
        pl.kernel, mesh=mesh,
        out_type=jax.ShapeDtypeStruct((e_num, _HP), jnp.float32),
        scratch_types=[
            pltpu.VMEM((span, ), jnp.int32),
            pltpu.VMEM((span, ), jnp.int32),
            pltpu.VMEM((2, _EC, _HPG), jnp.float32),
            pltpu.VMEM((2, _EC, _HPG), jnp.float32),
            pltpu.VMEM((2, _EC, _HP), jnp.float32),
            pltpu.SemaphoreType.DMA, pltpu.SemaphoreType.DMA,
            pltpu.SemaphoreType.DMA, pltpu.SemaphoreType.DMA,
            pltpu.SemaphoreType.DMA, pltpu.SemaphoreType.DMA,
        ],
    )
    def gather_sub(g_hbm, b_hbm, src_hbm, dst_hbm, e1_hbm,
                   sidx, didx, gbuf, bbuf, obuf,
                   sg0, sg1, sb0, sb1, so0, so1):
        nc = 2
        wid = lax.axis_index("s") * nc + lax.axis_index("c")
        tbase = pl.multiple_of(wid * span, 8)
        pltpu.sync_copy(src_hbm.at[pl.ds(tbase, span)], sidx)
        pltpu.sync_copy(dst_hbm.at[pl.ds(tbase, span)], didx)
        sg = (sg0, sg1)
        sb = (sb0, sb1)
        so = (so0, so1)

        def loff(c):
            return pl.multiple_of(
                jnp.minimum(c * _EC, span - _EC), 8)

        def start_gather(c, p):
            o = loff(c)
            pltpu.async_copy(g_hbm.at[sidx.at[pl.ds(o, _EC)]],
                             gbuf.at[p], sg[p])
            pltpu.async_copy(b_hbm.at[didx.at[pl.ds(o, _EC)]],
                             bbuf.at[p], sb[p])

        def wait_gather(p):
            pltpu.make_async_copy(g_hbm.at[pl.ds(0, _EC)], gbuf.at[p],
                                  sg[p]).wait()
            pltpu.make_async_copy(b_hbm.at[pl.ds(0, _EC)], bbuf.at[p],
                                  sb[p]).wait()

        def body_one(c, p):
            wait_gather(p)

            @pl.when(c >= 2)
            def _():
                pltpu.make_async_copy(e1_hbm.at[pl.ds(0, _EC)], obuf.at[p],
                                      so[p]).wait()

            def edge_body(i, _):
                for k in range(_HP // 16):
                    sl = pl.ds(k * 16, 16)
                    obuf[p, i, sl] = jnp.maximum(
                        gbuf[p, i, sl] - bbuf[p, i, sl], 0.0)
                return 0
            lax.fori_loop(0, _EC, edge_body, 0)
            pltpu.async_copy(obuf.at[p], e1_hbm.at[pl.ds(tbase + loff(c), _EC)],
                             so[p])
            start_gather(c + 2, p)

        start_gather(jnp.int32(0), 0)
        start_gather(jnp.int32(1), 1)

        def pair_body(i, _):
            body_one(2 * i, 0)
            body_one(2 * i + 1, 1)
            return 0
        lax.fori_loop(0, nch2 // 2, pair_body, 0)
        for p in (0, 1):
            wait_gather(p)
            pltpu.make_async_copy(e1_hbm.at[pl.ds(0, _EC)], obuf.at[p],
                                  so[p]).wait()

    return gather_sub


def _make_scatter_max(e_num, n_pad):
    mesh = plsc.VectorSubcoreMesh(core_axis_name="c", subcore_axis_name="s")

    nstage = 64

    @functools.partial(
        pl.kernel, mesh=mesh,
        out_type=jax.ShapeDtypeStruct((n_pad, _HP), jnp.float32),
        scratch_types=[
            pltpu.VMEM((nstage, _HP), jnp.float32),
            pltpu.VMEM((_EC, _HP), jnp.float32),
            pltpu.VMEM((_EC + 16,), jnp.int32),
            pltpu.VMEM((_NW + 16,), jnp.int32),
            pltpu.VMEM((_NW + 16,), jnp.int32),
        ],
    )
    def scatter_max(e2_hbm, dst_hbm, elo_hbm, ehi_hbm, out_hbm,
                    stage, ebuf, dbuf, elo_v, ehi_v):
        nc = 2
        wid = lax.axis_index("s") * nc + lax.axis_index("c")
        nlo = pl.multiple_of(wid * _NTILE, _NTILE)
        pltpu.sync_copy(elo_hbm, elo_v.at[pl.ds(0, _NW)])
        pltpu.sync_copy(ehi_hbm, ehi_v.at[pl.ds(0, _NW)])
        elo = elo_v[pl.ds(wid, 16)][0]
        ehi = ehi_v[pl.ds(wid, 16)][0]

        def zero_stage():
            def zero_row(r, _):
                for k in range(_HP // 16):
                    stage[r, pl.ds(k * 16, 16)] = jnp.zeros((16,), jnp.float32)
                return 0
            lax.fori_loop(0, nstage, zero_row, 0)

        def flush(cb):
            pltpu.sync_copy(
                stage, out_hbm.at[pl.ds(pl.multiple_of(nlo + cb, 8), nstage)])
            zero_stage()

        zero_stage()
        nchunks = (ehi - elo + _EC - 1) // _EC

        def chunk_body(j, cur_base):
            start = pl.multiple_of(jnp.minimum(elo + j * _EC, e_num - _EC), 8)
            pltpu.sync_copy(e2_hbm.at[pl.ds(start, _EC)], ebuf)
            pltpu.sync_copy(dst_hbm.at[pl.ds(start, _EC)],
                            dbuf.at[pl.ds(0, _EC)])

            def edge_body(i, cb):
                row = dbuf[pl.ds(i, 16)][0] - nlo
                tgt = jnp.clip((row // nstage) * nstage, 0, _NTILE - nstage)
                nflush = jnp.maximum((tgt - cb) // nstage, 0)

                def adv_body(m, c):
                    flush(c)
                    return c + nstage
                cb = lax.fori_loop(0, nflush, adv_body, cb)

                @pl.when((row >= cb) & (row < cb + nstage))
                def _():
                    r = row - cb
                    for k in range(_HP // 16):
                        sl = pl.ds(k * 16, 16)
                        stage[r, sl] = jnp.maximum(stage[r, sl],
                                                   ebuf[i, sl])
                return cb
            return lax.fori_loop(0, _EC, edge_body, cur_base)
        cur_base = lax.fori_loop(0, nchunks, chunk_body, 0)

        def tail_body(m, c):
            flush(c)
            return c + nstage
        lax.fori_loop(0, (_NTILE - cur_base) // nstage, tail_body, cur_base)

    return scatter_max


def kernel(x, xyz, edge_index, Wo1, bo1, Wo2, bo2, We1, be1, We2, be2,
           Wu1, bu1, Wu2, bu2):
    src = edge_index[0]
    dst = edge_index[1]
    n, d = x.shape
    e_num = src.shape[0]
    num_layers = Wo1.shape[0]
    n_pad = _NW * _NTILE

    order = jnp.argsort(dst)
    dst_s = dst[order]
    src_s = src[order]
    bounds = jnp.searchsorted(dst_s, jnp.arange(_NW + 1, dtype=jnp.int32) * _NTILE)
    elo8 = ((bounds[:_NW] // 8) * 8).astype(jnp.int32)
    ehi = bounds[1:].astype(jnp.int32)

    pad_c = _HP - We2.shape[-1]
    pad_g = _HPG - We2.shape[-1]
    We1p = jnp.pad(We1, ((0, 0), (0, 0), (0, pad_g)))
    be1p = jnp.pad(be1, ((0, 0), (0, pad_g)))
    We2p = jnp.pad(We2, ((0, 0), (0, pad_c), (0, pad_c)))
    be2p = jnp.pad(be2, ((0, 0), (0, pad_c)))
    Wu1p = jnp.pad(Wu1, ((0, 0), (0, pad_c), (0, pad_c)))
    bu1p = jnp.pad(bu1, ((0, 0), (0, pad_c)))
    Wu2p = jnp.pad(Wu2, ((0, 0), (0, pad_c), (0, 0)))

    BN = 400
    BE = 2000

    h = x
    full = lambda shape: pl.BlockSpec(shape, lambda i: (0,) * len(shape))

    node_call = pl.pallas_call(
        _node_kernel,
        grid=(n // BN,),
        in_specs=[
            pl.BlockSpec((BN, d), lambda i: (i, 0)),
            pl.BlockSpec((BN, 3), lambda i: (i, 0)),
            full(Wo1.shape[1:]), full((1, bo1.shape[-1])),
            full(Wo2.shape[1:]), full((1, 3)),
            full((3, _HPG)), full((d, _HPG)), full((1, _HPG)),
        ],
        out_specs=[
            pl.BlockSpec((BN, _HPG), lambda i: (i, 0)),
            pl.BlockSpec((BN, _HPG), lambda i: (i, 0)),
        ],
        out_shape=[
            jax.ShapeDtypeStruct((n, _HPG), jnp.float32),
            jax.ShapeDtypeStruct((n, _HPG), jnp.float32),
        ],
    )

    edge_call = pl.pallas_call(
        _edge_kernel,
        grid=(e_num // BE,),
        in_specs=[
            pl.BlockSpec((BE, _HP), lambda i: (i, 0)),
            full((_HP, _HP)), full((1, _HP)),
        ],
        out_specs=pl.BlockSpec((BE, _HP), lambda i: (i, 0)),
        out_shape=jax.ShapeDtypeStruct((e_num, _HP), jnp.float32),
    )

    update_call = pl.pallas_call(
        _update_kernel,
        grid=(n // BN,),
        in_specs=[
            pl.BlockSpec((BN, d), lambda i: (i, 0)),
            pl.BlockSpec((BN, _HP), lambda i: (i, 0)),
            full((_HP, _HP)), full((1, _HP)),
            full((_HP, d)), full((1, d)),
        ],
        out_specs=pl.BlockSpec((BN, d), lambda i: (i, 0)),
        out_shape=jax.ShapeDtypeStruct((n, d), jnp.float32),
    )

    scatter_call = _make_scatter_max(e_num, n_pad)
    gather_call = _make_gather_sub(e_num)

    for l in range(num_layers):
        g, b = node_call(
            h, xyz, Wo1[l], bo1[l][None], Wo2[l], bo2[l][None],
            We1p[l, :3], We1p[l, 3:], be1p[l][None])
        e1 = gather_call(g, b, src_s, dst_s)
        e2 = edge_call(e1, We2p[l], be2p[l][None])
        agg = scatter_call(e2, dst_s, elo8, ehi)[:n]
        h = update_call(h, agg, Wu1p[l], bu1p[l][None], Wu2p[l], bu2[l][None])
    return h


# speedup vs baseline: 1.5214x; 1.1810x over previous
import functools

import jax
import jax.numpy as jnp
from jax import lax
from jax.experimental import pallas as pl
from jax.experimental.pallas import tpu as pltpu
from jax.experimental.pallas import tpu_sc as plsc

_HP = 304
_HPG = 384
_NTILE = 320
_EC = 32
_NW = 32


def _node_kernel(h_ref, xyz_ref, wo1_ref, bo1_ref, wo2_ref, bo2_ref,
                 we1c_ref, we1h_ref, be1_ref, g_ref, b_ref):
    h = h_ref[...]
    xyz = xyz_ref[...]
    t = jnp.maximum(
        jnp.dot(h, wo1_ref[...], preferred_element_type=jnp.float32)
        + bo1_ref[...], 0.0)
    off = jnp.dot(t, wo2_ref[...], preferred_element_type=jnp.float32) + bo2_ref[...]
    c = xyz + off
    we1c = we1c_ref[...]
    g_ref[...] = (jnp.dot(h, we1h_ref[...], preferred_element_type=jnp.float32)
                  + jnp.dot(xyz, we1c, preferred_element_type=jnp.float32)
                  + be1_ref[...])
    b_ref[...] = jnp.dot(c, we1c, preferred_element_type=jnp.float32)


def _edge_kernel(e1_ref, we2_ref, be2_ref, e2_ref):
    e2_ref[...] = jnp.maximum(
        jnp.dot(e1_ref[...], we2_ref[...], preferred_element_type=jnp.float32)
        + be2_ref[...], 0.0)


def _update_kernel(h_ref, agg_ref, wu1_ref, bu1_ref, wu2_ref, bu2_ref, out_ref):
    t = jnp.maximum(
        jnp.dot(agg_ref[...], wu1_ref[...], preferred_element_type=jnp.float32)
        + bu1_ref[...], 0.0)
    out_ref[...] = h_ref[...] + (
        jnp.dot(t, wu2_ref[...], preferred_element_type=jnp.float32)
        + bu2_ref[...])


def _make_gather_sub(e_num):
    span = e_num // _NW
    nch = (span + _EC - 1) // _EC
    nch2 = nch + (nch % 2)
    mesh = plsc.VectorSubcoreMesh(core_axis_name="c", subcore_axis_name="s")

    @functools.partial(
        pl.kernel, mesh=mesh,
        out_type=jax.ShapeDtypeStruct((e_num, _HP), jnp.float32),
        scratch_types=[
            pltpu.VMEM((span, ), jnp.int32),
            pltpu.VMEM((span, ), jnp.int32),
            pltpu.VMEM((2, _EC, _HPG), jnp.float32),
            pltpu.VMEM((2, _EC, _HPG), jnp.float32),
            pltpu.VMEM((2, _EC, _HP), jnp.float32),
            pltpu.SemaphoreType.DMA, pltpu.SemaphoreType.DMA,
            pltpu.SemaphoreType.DMA, pltpu.SemaphoreType.DMA,
            pltpu.SemaphoreType.DMA, pltpu.SemaphoreType.DMA,
        ],
    )
    def gather_sub(g_hbm, b_hbm, src_hbm, dst_hbm, e1_hbm,
                   sidx, didx, gbuf, bbuf, obuf,
                   sg0, sg1, sb0, sb1, so0, so1):
        nc = 2
        wid = lax.axis_index("s") * nc + lax.axis_index("c")
        tbase = pl.multiple_of(wid * span, 8)
        pltpu.sync_copy(src_hbm.at[pl.ds(tbase, span)], sidx)
        pltpu.sync_copy(dst_hbm.at[pl.ds(tbase, span)], didx)
        sg = (sg0, sg1)
        sb = (sb0, sb1)
        so = (so0, so1)

        def loff(c):
            return pl.multiple_of(
                jnp.minimum(c * _EC, span - _EC), 8)

        def start_gather(c, p):
            o = loff(c)
            pltpu.async_copy(g_hbm.at[sidx.at[pl.ds(o, _EC)]],
                             gbuf.at[p], sg[p])
            pltpu.async_copy(b_hbm.at[didx.at[pl.ds(o, _EC)]],
                             bbuf.at[p], sb[p])

        def wait_gather(p):
            pltpu.make_async_copy(g_hbm.at[pl.ds(0, _EC)], gbuf.at[p],
                                  sg[p]).wait()
            pltpu.make_async_copy(b_hbm.at[pl.ds(0, _EC)], bbuf.at[p],
                                  sb[p]).wait()

        def body_one(c, p):
            wait_gather(p)

            @pl.when(c >= 2)
            def _():
                pltpu.make_async_copy(e1_hbm.at[pl.ds(0, _EC)], obuf.at[p],
                                      so[p]).wait()

            def edge_body(i, _):
                for k in range(_HP // 16):
                    sl = pl.ds(k * 16, 16)
                    obuf[p, i, sl] = jnp.maximum(
                        gbuf[p, i, sl] - bbuf[p, i, sl], 0.0)
                return 0
            lax.fori_loop(0, _EC, edge_body, 0)
            pltpu.async_copy(obuf.at[p], e1_hbm.at[pl.ds(tbase + loff(c), _EC)],
                             so[p])
            start_gather(c + 2, p)

        start_gather(jnp.int32(0), 0)
        start_gather(jnp.int32(1), 1)

        def pair_body(i, _):
            body_one(2 * i, 0)
            body_one(2 * i + 1, 1)
            return 0
        lax.fori_loop(0, nch2 // 2, pair_body, 0)
        for p in (0, 1):
            wait_gather(p)
            pltpu.make_async_copy(e1_hbm.at[pl.ds(0, _EC)], obuf.at[p],
                                  so[p]).wait()

    return gather_sub


def _make_scatter_max(e_num, n_pad):
    mesh = plsc.VectorSubcoreMesh(core_axis_name="c", subcore_axis_name="s")

    nstage = 64
    log_ns = 6

    @functools.partial(
        pl.kernel, mesh=mesh,
        out_type=jax.ShapeDtypeStruct((n_pad, _HP), jnp.float32),
        scratch_types=[
            pltpu.VMEM((nstage, _HP), jnp.float32),
            pltpu.VMEM((2, _EC, _HP), jnp.float32),
            pltpu.VMEM((2, _EC + 16), jnp.int32),
            pltpu.VMEM((_NW + 16,), jnp.int32),
            pltpu.VMEM((_NW + 16,), jnp.int32),
            pltpu.SemaphoreType.DMA, pltpu.SemaphoreType.DMA,
            pltpu.SemaphoreType.DMA, pltpu.SemaphoreType.DMA,
        ],
    )
    def scatter_max(e2_hbm, dst_hbm, elo_hbm, ehi_hbm, out_hbm,
                    stage, ebuf, dbuf, elo_v, ehi_v, se0, se1, sd0, sd1):
        nc = 2
        wid = lax.axis_index("s") * nc + lax.axis_index("c")
        nlo = pl.multiple_of(wid * _NTILE, _NTILE)
        pltpu.sync_copy(elo_hbm, elo_v.at[pl.ds(0, _NW)])
        pltpu.sync_copy(ehi_hbm, ehi_v.at[pl.ds(0, _NW)])
        elo = elo_v[pl.ds(wid, 16)][0]
        ehi = ehi_v[pl.ds(wid, 16)][0]
        se = (se0, se1)
        sd = (sd0, sd1)

        def zero_stage():
            def zero_row(r, _):
                for k in range(_HP // 16):
                    stage[r, pl.ds(k * 16, 16)] = jnp.zeros((16,), jnp.float32)
                return 0
            lax.fori_loop(0, nstage, zero_row, 0)

        def flush(cb):
            pltpu.sync_copy(
                stage, out_hbm.at[pl.ds(pl.multiple_of(nlo + cb, 8), nstage)])
            zero_stage()

        def start_chunk(c, p):
            start = pl.multiple_of(
                jnp.minimum(elo + c * _EC, e_num - _EC), 8)
            pltpu.async_copy(e2_hbm.at[pl.ds(start, _EC)], ebuf.at[p], se[p])
            pltpu.async_copy(dst_hbm.at[pl.ds(start, _EC)],
                             dbuf.at[p, pl.ds(0, _EC)], sd[p])

        def wait_chunk(p):
            pltpu.make_async_copy(e2_hbm.at[pl.ds(0, _EC)], ebuf.at[p],
                                  se[p]).wait()
            pltpu.make_async_copy(dst_hbm.at[pl.ds(0, _EC)],
                                  dbuf.at[p, pl.ds(0, _EC)], sd[p]).wait()

        zero_stage()
        nch = (ehi - elo + _EC - 1) >> 5
        start_chunk(jnp.int32(0), 0)
        start_chunk(jnp.int32(1), 1)

        def body_one(c, p, cb):
            wait_chunk(p)

            def edge_body(i, cb):
                row = dbuf[p, pl.ds(i, 16)][0] - nlo
                tgt = jnp.clip((row >> log_ns) << log_ns, 0, _NTILE - nstage)
                nflush = jnp.maximum((tgt - cb) >> log_ns, 0)

                def adv_body(m, c2):
                    flush(c2)
                    return c2 + nstage
                cb = lax.fori_loop(0, nflush, adv_body, cb)

                @pl.when((row >= cb) & (row < cb + nstage))
                def _():
                    r = row - cb
                    for k in range(_HP // 16):
                        sl = pl.ds(k * 16, 16)
                        stage[r, sl] = jnp.maximum(stage[r, sl],
                                                   ebuf[p, i, sl])
                return cb
            cb = lax.fori_loop(0, _EC, edge_body, cb)
            start_chunk(jnp.minimum(c + 2, jnp.maximum(nch - 1, 0)), p)
            return cb

        def pair_body(i, cb):
            cb = body_one(2 * i, 0, cb)
            cb = body_one(2 * i + 1, 1, cb)
            return cb
        cur_base = lax.fori_loop(0, (nch + 1) >> 1, pair_body, 0)
        for p in (0, 1):
            wait_chunk(p)

        def tail_body(m, c):
            flush(c)
            return c + nstage
        lax.fori_loop(0, (_NTILE - cur_base) >> log_ns, tail_body, cur_base)

    return scatter_max


def kernel(x, xyz, edge_index, Wo1, bo1, Wo2, bo2, We1, be1, We2, be2,
           Wu1, bu1, Wu2, bu2):
    src = edge_index[0]
    dst = edge_index[1]
    n, d = x.shape
    e_num = src.shape[0]
    num_layers = Wo1.shape[0]
    n_pad = _NW * _NTILE

    order = jnp.argsort(dst)
    dst_s = dst[order]
    src_s = src[order]
    bounds = jnp.searchsorted(dst_s, jnp.arange(_NW + 1, dtype=jnp.int32) * _NTILE)
    elo8 = ((bounds[:_NW] // 8) * 8).astype(jnp.int32)
    ehi = bounds[1:].astype(jnp.int32)

    pad_c = _HP - We2.shape[-1]
    pad_g = _HPG - We2.shape[-1]
    We1p = jnp.pad(We1, ((0, 0), (0, 0), (0, pad_g)))
    be1p = jnp.pad(be1, ((0, 0), (0, pad_g)))
    We2p = jnp.pad(We2, ((0, 0), (0, pad_c), (0, pad_c)))
    be2p = jnp.pad(be2, ((0, 0), (0, pad_c)))
    Wu1p = jnp.pad(Wu1, ((0, 0), (0, pad_c), (0, pad_c)))
    bu1p = jnp.pad(bu1, ((0, 0), (0, pad_c)))
    Wu2p = jnp.pad(Wu2, ((0, 0), (0, pad_c), (0, 0)))

    BN = 400
    BE = 2000

    h = x
    full = lambda shape: pl.BlockSpec(shape, lambda i: (0,) * len(shape))

    node_call = pl.pallas_call(
        _node_kernel,
        grid=(n // BN,),
        in_specs=[
            pl.BlockSpec((BN, d), lambda i: (i, 0)),
            pl.BlockSpec((BN, 3), lambda i: (i, 0)),
            full(Wo1.shape[1:]), full((1, bo1.shape[-1])),
            full(Wo2.shape[1:]), full((1, 3)),
            full((3, _HPG)), full((d, _HPG)), full((1, _HPG)),
        ],
        out_specs=[
            pl.BlockSpec((BN, _HPG), lambda i: (i, 0)),
            pl.BlockSpec((BN, _HPG), lambda i: (i, 0)),
        ],
        out_shape=[
            jax.ShapeDtypeStruct((n, _HPG), jnp.float32),
            jax.ShapeDtypeStruct((n, _HPG), jnp.float32),
        ],
    )

    edge_call = pl.pallas_call(
        _edge_kernel,
        grid=(e_num // BE,),
        in_specs=[
            pl.BlockSpec((BE, _HP), lambda i: (i, 0)),
            full((_HP, _HP)), full((1, _HP)),
        ],
        out_specs=pl.BlockSpec((BE, _HP), lambda i: (i, 0)),
        out_shape=jax.ShapeDtypeStruct((e_num, _HP), jnp.float32),
    )

    update_call = pl.pallas_call(
        _update_kernel,
        grid=(n // BN,),
        in_specs=[
            pl.BlockSpec((BN, d), lambda i: (i, 0)),
            pl.BlockSpec((BN, _HP), lambda i: (i, 0)),
            full((_HP, _HP)), full((1, _HP)),
            full((_HP, d)), full((1, d)),
        ],
        out_specs=pl.BlockSpec((BN, d), lambda i: (i, 0)),
        out_shape=jax.ShapeDtypeStruct((n, d), jnp.float32),
    )

    scatter_call = _make_scatter_max(e_num, n_pad)
    gather_call = _make_gather_sub(e_num)

    for l in range(num_layers):
        g, b = node_call(
            h, xyz, Wo1[l], bo1[l][None], Wo2[l], bo2[l][None],
            We1p[l, :3], We1p[l, 3:], be1p[l][None])
        e1 = gather_call(g, b, src_s, dst_s)
        e2 = edge_call(e1, We2p[l], be2p[l][None])
        agg = scatter_call(e2, dst_s, elo8, ehi)[:n]
        h = update_call(h, agg, Wu1p[l], bu1p[l][None], Wu2p[l], bu2[l][None])
    return h
